# baseline (speedup 1.0000x reference)
import functools

import jax
import jax.numpy as jnp
from jax import lax
from jax.experimental import pallas as pl
from jax.experimental.pallas import tpu as pltpu
from jax.experimental.pallas import tpu_sc as plsc

_B = 4096
_H = 200
_D = 32
_BL = 128


def _gather_kernel(num_cores):
    @functools.partial(
        pl.kernel,
        mesh=plsc.VectorSubcoreMesh(core_axis_name="c", subcore_axis_name="s"),
        out_type=jax.ShapeDtypeStruct((_H, _D, _B), jnp.float32),
        compiler_params=pltpu.CompilerParams(needs_layout_passes=False),
        scratch_types=[
            pltpu.VMEM((_BL * _H,), jnp.int32),
            pltpu.VMEM((2, _BL), jnp.int32),
            pltpu.VMEM((2, _BL), jnp.int32),
            pltpu.VMEM((2, _BL, 128), jnp.float32),
            pltpu.VMEM((2, _D, _BL), jnp.float32),
            pltpu.SemaphoreType.DMA,
            pltpu.SemaphoreType.DMA,
        ],
    )
    def k(table_hbm, idx_hbm, out_hbm, raw_v, hi_v, p32_v, quad_v, oblk_v,
          gsem, osem):
        wid = lax.axis_index("s") * num_cores + lax.axis_index("c")
        b0 = wid * _BL
        iota = lax.iota(jnp.int32, 16)

        pltpu.sync_copy(idx_hbm.at[pl.ds(wid * (_BL * _H), _BL * _H)], raw_v)

        def prep(h, p):
            for blk in range(8):
                pos = iota * _H + (blk * 16 * _H + h)
                v = plsc.load_gather(raw_v, [pos])
                hi_v[p, pl.ds(blk * 16, 16)] = lax.shift_right_logical(v, 2)
                p32_v[p, pl.ds(blk * 16, 16)] = lax.shift_left(
                    lax.bitwise_and(v, 3), 5)

        def fire_gather(p):
            pltpu.async_copy(table_hbm.at[hi_v.at[p]], quad_v.at[p], gsem)

        def wait_gather(p):
            pltpu.make_async_copy(
                table_hbm.at[pl.ds(0, _BL)], quad_v.at[p], gsem).wait()

        def unpack(p):
            for blk in range(8):
                rows = iota + blk * 16
                pbase = p32_v[p, pl.ds(blk * 16, 16)]
                for c in range(_D):
                    v = plsc.load_gather(quad_v.at[p], [rows, pbase + c])
                    oblk_v[p, c, pl.ds(blk * 16, 16)] = v

        def fire_write(h, p):
            pltpu.async_copy(
                oblk_v.at[p], out_hbm.at[h, :, pl.ds(b0, _BL)], osem)

        def wait_write(p):
            pltpu.make_async_copy(
                out_hbm.at[0, :, pl.ds(0, _BL)], oblk_v.at[p], osem).wait()

        prep(0, 0)
        prep(1, 1)
        fire_gather(0)
        fire_gather(1)

        def body(i, carry):
            for p in range(2):
                h = 2 * i + p
                wait_gather(p)

                @pl.when(i > 0)
                def _():
                    wait_write(p)

                unpack(p)

                @pl.when(i < (_H // 2) - 1)
                def _():
                    prep(h + 2, p)
                    fire_gather(p)

                fire_write(h, p)
            return carry

        lax.fori_loop(0, _H // 2, body, 0)
        wait_write(0)
        wait_write(1)

    return k


def kernel(input, lengths, emb_table):
    b, h = input.shape
    v, d = emb_table.shape
    info = plsc.get_sparse_core_info()
    idx = input.reshape(b * h)
    table128 = emb_table.reshape(v // 4, 128)
    out_p = _gather_kernel(info.num_cores)(table128, idx)
    return (out_p.transpose(2, 0, 1), lengths)


# speedup vs baseline: 1.1942x; 1.1942x over previous
import functools

import jax
import jax.numpy as jnp
from jax import lax
from jax.experimental import pallas as pl
from jax.experimental.pallas import tpu as pltpu
from jax.experimental.pallas import tpu_sc as plsc

_B = 4096
_H = 200
_D = 32
_BL = 128


def _gather_kernel(num_cores):
    @functools.partial(
        pl.kernel,
        mesh=plsc.VectorSubcoreMesh(core_axis_name="c", subcore_axis_name="s"),
        out_type=jax.ShapeDtypeStruct((_H, _D, _B), jnp.float32),
        compiler_params=pltpu.CompilerParams(needs_layout_passes=False),
        scratch_types=[
            pltpu.VMEM((_BL * _H,), jnp.int32),
            pltpu.VMEM((2, _BL), jnp.int32),
            pltpu.VMEM((2, _BL), jnp.int32),
            pltpu.VMEM((2, _BL, 128), jnp.float32),
            pltpu.VMEM((2, _D, _BL), jnp.float32),
            pltpu.SemaphoreType.DMA,
            pltpu.SemaphoreType.DMA,
        ],
    )
    def k(table_hbm, idx_hbm, out_hbm, raw_v, hi_v, p32_v, quad_v, oblk_v,
          gsem, osem):
        wid = lax.axis_index("s") * num_cores + lax.axis_index("c")
        b0 = wid * _BL
        iota = lax.iota(jnp.int32, 16)
        iota_h = iota * _H

        pltpu.sync_copy(idx_hbm.at[pl.ds(wid * (_BL * _H), _BL * _H)], raw_v)

        def prep(h, p):
            for blk in range(8):
                v = plsc.load_gather(raw_v, [iota_h + (blk * 16 * _H + h)])
                hi_v[p, pl.ds(blk * 16, 16)] = lax.shift_right_logical(v, 2)
                p32_v[p, pl.ds(blk * 16, 16)] = lax.shift_left(
                    lax.bitwise_and(v, 3), 5)

        def fire_gather(p):
            pltpu.async_copy(table_hbm.at[hi_v.at[p]], quad_v.at[p], gsem)

        def wait_gather(p):
            pltpu.make_async_copy(
                table_hbm.at[pl.ds(0, _BL)], quad_v.at[p], gsem).wait()

        def unpack(p):
            pending = None
            for blk in range(8):
                r16 = iota + blk * 16
                pbase = p32_v[p, pl.ds(blk * 16, 16)]
                for c in range(_D):
                    v = plsc.load_gather(quad_v.at[p], [r16, pbase + c])
                    if pending is not None:
                        pv, pc, pblk = pending
                        oblk_v[p, pc, pl.ds(pblk * 16, 16)] = pv
                    pending = (v, c, blk)
            pv, pc, pblk = pending
            oblk_v[p, pc, pl.ds(pblk * 16, 16)] = pv

        def fire_write(h, p):
            pltpu.async_copy(
                oblk_v.at[p], out_hbm.at[h, :, pl.ds(b0, _BL)], osem)

        def wait_write(p):
            pltpu.make_async_copy(
                out_hbm.at[0, :, pl.ds(0, _BL)], oblk_v.at[p], osem).wait()

        prep(0, 0)
        prep(1, 1)
        fire_gather(0)
        fire_gather(1)

        def body(i, carry):
            for p in range(2):
                h = 2 * i + p
                wait_gather(p)

                @pl.when(i > 0)
                def _():
                    wait_write(p)

                unpack(p)

                @pl.when(i < (_H // 2) - 1)
                def _():
                    prep(h + 2, p)
                    fire_gather(p)

                fire_write(h, p)
            return carry

        lax.fori_loop(0, _H // 2, body, 0)
        wait_write(0)
        wait_write(1)

    return k


def kernel(input, lengths, emb_table):
    b, h = input.shape
    v, d = emb_table.shape
    info = plsc.get_sparse_core_info()
    idx = input.reshape(b * h)
    table128 = emb_table.reshape(v // 4, 128)
    out_p = _gather_kernel(info.num_cores)(table128, idx)
    return (out_p.transpose(2, 0, 1), lengths)


# speedup vs baseline: 1.3522x; 1.1323x over previous
import functools

import jax
import jax.numpy as jnp
from jax import lax
from jax.experimental import pallas as pl
from jax.experimental.pallas import tpu as pltpu
from jax.experimental.pallas import tpu_sc as plsc

_B = 4096
_H = 200
_D = 32
_BL = 128


def _gather_kernel(num_cores):
    @functools.partial(
        pl.kernel,
        mesh=plsc.VectorSubcoreMesh(core_axis_name="c", subcore_axis_name="s"),
        out_type=jax.ShapeDtypeStruct((_H, _D, _B), jnp.float32),
        compiler_params=pltpu.CompilerParams(needs_layout_passes=False),
        scratch_types=[
            pltpu.VMEM((_BL * _H,), jnp.int32),
            pltpu.VMEM((2, _BL), jnp.int32),
            pltpu.VMEM((2, _BL), jnp.int32),
            pltpu.VMEM((2, _BL, 128), jnp.float32),
            pltpu.VMEM((2, _D, _BL), jnp.float32),
            pltpu.SemaphoreType.DMA,
            pltpu.SemaphoreType.DMA,
        ],
    )
    def k(table_hbm, idx_hbm, out_hbm, raw_v, hi_v, p32_v, quad_v, oblk_v,
          gsem, osem):
        wid = lax.axis_index("s") * num_cores + lax.axis_index("c")
        b0 = wid * _BL
        iota = lax.iota(jnp.int32, 16)
        iota_h = iota * _H

        pltpu.sync_copy(idx_hbm.at[pl.ds(wid * (_BL * _H), _BL * _H)], raw_v)

        def prep(h, p):
            for blk in range(8):
                v = plsc.load_gather(raw_v, [iota_h + (blk * 16 * _H + h)])
                hi_v[p, pl.ds(blk * 16, 16)] = lax.shift_right_logical(v, 2)
                p32_v[p, pl.ds(blk * 16, 16)] = lax.shift_left(
                    lax.bitwise_and(v, 3), 5)

        def fire_gather(p):
            pltpu.async_copy(table_hbm.at[hi_v.at[p]], quad_v.at[p], gsem)

        def wait_gather(p):
            pltpu.make_async_copy(
                table_hbm.at[pl.ds(0, _BL)], quad_v.at[p], gsem).wait()

        def unpack(p):
            ob2 = oblk_v.at[p]
            pending = None
            for blk in range(8):
                r16 = iota + blk * 16
                pbase = p32_v[p, pl.ds(blk * 16, 16)]
                for c in range(_D):
                    ksw = lax.bitwise_and(iota + c, _D - 1)
                    v = plsc.load_gather(quad_v.at[p], [r16, pbase + ksw])
                    if pending is not None:
                        pv, pk, pr = pending
                        plsc.store_scatter(ob2, [pk, pr], pv)
                    pending = (v, ksw, r16)
            pv, pk, pr = pending
            plsc.store_scatter(ob2, [pk, pr], pv)

        def fire_write(h, p):
            pltpu.async_copy(
                oblk_v.at[p], out_hbm.at[h, :, pl.ds(b0, _BL)], osem)

        def wait_write(p):
            pltpu.make_async_copy(
                out_hbm.at[0, :, pl.ds(0, _BL)], oblk_v.at[p], osem).wait()

        prep(0, 0)
        prep(1, 1)
        fire_gather(0)
        fire_gather(1)

        def body(i, carry):
            for p in range(2):
                h = 2 * i + p
                wait_gather(p)

                @pl.when(i > 0)
                def _():
                    wait_write(p)

                unpack(p)

                @pl.when(i < (_H // 2) - 1)
                def _():
                    prep(h + 2, p)
                    fire_gather(p)

                fire_write(h, p)
            return carry

        lax.fori_loop(0, _H // 2, body, 0)
        wait_write(0)
        wait_write(1)

    return k


def kernel(input, lengths, emb_table):
    b, h = input.shape
    v, d = emb_table.shape
    info = plsc.get_sparse_core_info()
    idx = input.reshape(b * h)
    table128 = emb_table.reshape(v // 4, 128)
    out_p = _gather_kernel(info.num_cores)(table128, idx)
    return (out_p.transpose(2, 0, 1), lengths)
